# baseline (speedup 1.0000x reference)
import functools

import jax
import jax.numpy as jnp
from jax import lax
from jax.experimental import pallas as pl
from jax.experimental.pallas import tpu as pltpu
from jax.experimental.pallas import tpu_sc as plsc

NC = 2
NS = 16
NW = NC * NS
LANES = 16
SUB = 400
GCH = 80
NCH = 400

LOG2 = 0.6931471805599453
C1 = 0.5
C2 = -1.0 / 12.0
C3 = 1.0 / 45.0
C4 = -17.0 / 2520.0

_SC_PARAMS = pltpu.CompilerParams(
    needs_layout_passes=False, use_tc_tiling_on_sc=False
)

_ILV = plsc.PackFormat.INTERLEAVED


def _splat(vec, k):
    idx = jnp.full((LANES,), k, jnp.int32)
    return vec.at[idx].get(mode="promise_in_bounds")


def _mesh():
    return plsc.VectorSubcoreMesh(
        core_axis_name="c", subcore_axis_name="s", num_cores=NC, num_subcores=NS
    )


def _sc_precompute(emb, wfull, bfull):
    N, D = emb.shape
    NP = 2 * D // LANES
    nchunks = N // NCH
    nbat = NCH // 8

    @functools.partial(
        pl.kernel,
        out_type=[
            jax.ShapeDtypeStruct((N, D // 2), jnp.int32),
            jax.ShapeDtypeStruct((N, D // 2), jnp.int32),
        ],
        mesh=_mesh(),
        scratch_types=[
            pltpu.VMEM((NCH, D), jnp.float32),
            pltpu.VMEM((NCH, D // 2), jnp.int32),
            pltpu.VMEM((NCH, D // 2), jnp.int32),
            pltpu.VMEM((D, 2 * D), jnp.float32),
            pltpu.VMEM((2 * D,), jnp.float32),
            pltpu.VMEM((D, D // 2), jnp.int32),
            pltpu.VMEM((D, D // 2), jnp.int32),
        ],
        compiler_params=_SC_PARAMS,
    )
    def k(emb_hbm, w_hbm, b_hbm, a_hbm, bm_hbm, xb, aout, bout, wv, bv,
          wpa, wpb):
        wid = lax.axis_index("s") * NC + lax.axis_index("c")
        pltpu.sync_copy(w_hbm, wv)
        pltpu.sync_copy(b_hbm, bv)
        for kf in range(D):
            pa = plsc.pack(wv[kf, pl.ds(0, LANES)], wv[kf, pl.ds(LANES, LANES)],
                           format=_ILV)
            pb = plsc.pack(wv[kf, pl.ds(2 * LANES, LANES)],
                           wv[kf, pl.ds(3 * LANES, LANES)], format=_ILV)
            wpa[kf, :] = plsc.bitcast(pa, jnp.int32)
            wpb[kf, :] = plsc.bitcast(pb, jnp.int32)
        bpA = plsc.pack(bv[pl.ds(0, LANES)], bv[pl.ds(LANES, LANES)], format=_ILV)
        zpB = jnp.zeros((2 * LANES,), jnp.bfloat16)
        nme = (nchunks - 1 - wid) // NW + 1

        def chunk(i, carry):
            off = (wid + i * NW) * NCH
            pltpu.sync_copy(emb_hbm.at[pl.ds(off, NCH)], xb)

            def bat(bi, c2):
                vbase = bi * 8
                acca = [bpA for _ in range(8)]
                accb = [zpB for _ in range(8)]
                dws = []
                for v in range(8):
                    e0 = xb[vbase + v, pl.ds(0, LANES)]
                    e1 = xb[vbase + v, pl.ds(LANES, LANES)]
                    dws.append((
                        plsc.bitcast(plsc.pack(e0, e0, format=_ILV), jnp.int32),
                        plsc.bitcast(plsc.pack(e1, e1, format=_ILV), jnp.int32),
                    ))
                for kf in range(D):
                    wa = plsc.bitcast(wpa[kf, :], jnp.bfloat16)
                    wb = plsc.bitcast(wpb[kf, :], jnp.bfloat16)
                    for v in range(8):
                        es = plsc.bitcast(
                            _splat(dws[v][kf // LANES], kf % LANES), jnp.bfloat16)
                        acca[v] = acca[v] + es * wa
                        accb[v] = accb[v] + es * wb
                for v in range(8):
                    aout[vbase + v, :] = plsc.bitcast(acca[v], jnp.int32)
                    bout[vbase + v, :] = plsc.bitcast(accb[v], jnp.int32)
                return c2

            lax.fori_loop(0, nbat, bat, 0)
            pltpu.sync_copy(aout, a_hbm.at[pl.ds(off, NCH)])
            pltpu.sync_copy(bout, bm_hbm.at[pl.ds(off, NCH)])
            return carry

        lax.fori_loop(0, nme, chunk, 0)

    return k(emb, wfull, bfull)


def _sc_loss_partials(edges, labels, A2, B2, params):
    E = edges.shape[0]
    NWRD = A2.shape[1]
    per_w = E // NW
    nsub = per_w // SUB
    ngrp = SUB // LANES
    ngath = SUB // GCH
    assert nsub % 2 == 1 and nsub >= 3

    scratch = [
        pltpu.VMEM((SUB, 2), jnp.int32),
        pltpu.VMEM((SUB, 2), jnp.int32),
        pltpu.VMEM((SUB,), jnp.int32),
        pltpu.VMEM((SUB,), jnp.int32),
        pltpu.VMEM((SUB,), jnp.int32),
        pltpu.VMEM((SUB, NWRD), jnp.int32),
        pltpu.VMEM((SUB, NWRD), jnp.int32),
        pltpu.VMEM((SUB,), jnp.int32),
        pltpu.VMEM((SUB,), jnp.int32),
        pltpu.VMEM((SUB,), jnp.int32),
        pltpu.VMEM((SUB, NWRD), jnp.int32),
        pltpu.VMEM((SUB, NWRD), jnp.int32),
        pltpu.VMEM((64,), jnp.float32),
        pltpu.VMEM((LANES,), jnp.float32),
        pltpu.SemaphoreType.DMA,
        pltpu.SemaphoreType.DMA,
        pltpu.SemaphoreType.DMA,
        pltpu.SemaphoreType.DMA,
    ]

    @functools.partial(
        pl.kernel,
        out_type=jax.ShapeDtypeStruct((NW, LANES), jnp.float32),
        mesh=_mesh(),
        scratch_types=scratch,
        compiler_params=_SC_PARAMS,
    )
    def k(edges_hbm, labels_hbm, a_hbm, b_hbm, params_hbm, out_hbm,
          eb0, eb1,
          sidx0, didx0, lab0, arows0, brows0,
          sidx1, didx1, lab1, arows1, brows1,
          pvv, accv, semi0, semi1, semg0, semg1):
        wid = lax.axis_index("s") * NC + lax.axis_index("c")
        base = wid * per_w
        pltpu.sync_copy(params_hbm, pvv)
        dv0 = pvv[pl.ds(0, LANES)]
        dv1 = pvv[pl.ds(LANES, LANES)]
        t0v = _splat(pvv[pl.ds(2 * LANES, LANES)], 0)
        dpairs = [
            plsc.pack(_splat(dv0, w), _splat(dv1, w), format=_ILV)
            for w in range(NWRD)
        ]
        zero_bf = jnp.zeros((2 * LANES,), jnp.bfloat16)
        iota = lax.iota(jnp.int32, LANES)

        bufs = [
            (eb0, sidx0, didx0, lab0, arows0, brows0, semi0, semg0),
            (eb1, sidx1, didx1, lab1, arows1, brows1, semi1, semg1),
        ]
        zcol = jnp.zeros((LANES,), jnp.int32)
        ocol = jnp.ones((LANES,), jnp.int32)

        def idx_copies(buf, c):
            eb, _, _, lab, _, _, semi, _ = buf
            off = base + c * SUB
            return [
                pltpu.make_async_copy(edges_hbm.at[pl.ds(off, SUB)], eb, semi),
                pltpu.make_async_copy(labels_hbm.at[pl.ds(off, SUB)], lab, semi),
            ]

        def extract(buf):
            eb, sidx, didx = buf[0], buf[1], buf[2]

            def bld(g, carry):
                rows = iota + g * LANES
                sv = plsc.load_gather(eb, [rows, zcol])
                dv = plsc.load_gather(eb, [rows, ocol])
                o = pl.multiple_of(g * LANES, LANES)
                sidx[pl.ds(o, LANES)] = sv
                didx[pl.ds(o, LANES)] = dv
                return carry

            lax.fori_loop(0, ngrp, bld, 0)

        def start_idx(buf, c):
            for cp in idx_copies(buf, c):
                cp.start()

        def wait_idx(buf, c):
            for cp in idx_copies(buf, c):
                cp.wait()

        def g_copies(buf):
            _, sidx, didx, _, arows, brows, _, semg = buf
            cps = []
            for i in range(ngath):
                sl = pl.ds(i * GCH, GCH)
                cps.append(pltpu.make_async_copy(a_hbm.at[sidx.at[sl]], arows.at[sl], semg))
                cps.append(pltpu.make_async_copy(b_hbm.at[didx.at[sl]], brows.at[sl], semg))
            return cps

        def fire_g(buf):
            for cp in g_copies(buf):
                cp.start()

        def wait_g(buf):
            for cp in g_copies(buf):
                cp.wait()

        def compute(buf, acc):
            _, _, _, lab, arows, brows, _, _ = buf

            def grp(g, acc2):
                rows = iota + g * LANES
                tl = t0v
                th = jnp.zeros((LANES,), jnp.float32)
                for w in range(NWRD):
                    col = jnp.full((LANES,), w, jnp.int32)
                    wa = plsc.load_gather(arows, [rows, col])
                    wb = plsc.load_gather(brows, [rows, col])
                    u = plsc.bitcast(wa, jnp.bfloat16) + plsc.bitcast(wb, jnp.bfloat16)
                    h = jnp.maximum(u, zero_bf)
                    pr = h * dpairs[w]
                    lo, hi = plsc.unpack(pr, format=_ILV)
                    tl = tl + lo
                    th = th + hi
                t = tl + th
                o = pl.multiple_of(g * LANES, LANES)
                lv = lab[pl.ds(o, LANES)]
                s = (1 - 2 * lv).astype(jnp.float32)
                z = jnp.exp(-jnp.abs(t))
                q = 0.5 * jnp.sign(t) * (1.0 - z) / (1.0 + z)
                q2 = q * q
                gq = LOG2 + q2 * (C1 + q2 * (C2 + q2 * (C3 + q2 * C4)))
                return acc2 + (gq - s * q)

            return lax.fori_loop(0, ngrp, grp, acc)

        start_idx(bufs[0], 0)
        wait_idx(bufs[0], 0)
        extract(bufs[0])
        fire_g(bufs[0])
        start_idx(bufs[1], 1)

        def body(cc, acc):
            c = 2 * cc
            c2, c3, c4 = c + 1, c + 2, c + 3
            wait_idx(bufs[1], c2)
            extract(bufs[1])
            fire_g(bufs[1])

            @pl.when(c3 < nsub)
            def _():
                start_idx(bufs[0], c3)

            wait_g(bufs[0])
            acc = compute(bufs[0], acc)

            @pl.when(c3 < nsub)
            def _():
                wait_idx(bufs[0], c3)
                extract(bufs[0])
                fire_g(bufs[0])

            @pl.when(c4 < nsub)
            def _():
                start_idx(bufs[1], c4)

            wait_g(bufs[1])
            acc = compute(bufs[1], acc)
            return acc

        acc = lax.fori_loop(0, (nsub - 1) // 2, body, jnp.zeros((LANES,), jnp.float32))
        wait_g(bufs[0])
        acc = compute(bufs[0], acc)

        accv[...] = acc
        pltpu.sync_copy(accv, out_hbm.at[wid])

    return k(edges, labels, A2, B2, params)


def kernel(edges, labels, word_embeddings, W1, b1, W2, b2):
    D = word_embeddings.shape[1]
    wfull = jnp.concatenate([W1[:D], W1[D:]], axis=1)
    bfull = jnp.concatenate([b1, jnp.zeros((D,), jnp.float32)])
    A2, B2 = _sc_precompute(word_embeddings, wfull, bfull)
    params = jnp.concatenate([
        W2[:, 0] - W2[:, 1],
        (b2[0] - b2[1])[None],
        jnp.zeros((63 - D,), jnp.float32),
    ])
    partials = _sc_loss_partials(edges, labels, A2, B2, params)
    return jnp.sum(partials) / edges.shape[0]


# speedup vs baseline: 5.6811x; 5.6811x over previous
import functools

import jax
import jax.numpy as jnp
from jax import lax
from jax.experimental import pallas as pl
from jax.experimental.pallas import tpu as pltpu
from jax.experimental.pallas import tpu_sc as plsc

NC = 2
NS = 16
NW = NC * NS
LANES = 16
SUB = 400
GCH = 80
NCH = 400

LOG2 = 0.6931471805599453
C1 = 0.5
C2 = -1.0 / 12.0
C3 = 1.0 / 45.0
C4 = -17.0 / 2520.0

_SC_PARAMS = pltpu.CompilerParams(
    needs_layout_passes=False, use_tc_tiling_on_sc=False
)

_ILV = plsc.PackFormat.INTERLEAVED


def _splat(vec, k):
    idx = jnp.full((LANES,), k, jnp.int32)
    return vec.at[idx].get(mode="promise_in_bounds")


def _mesh():
    return plsc.VectorSubcoreMesh(
        core_axis_name="c", subcore_axis_name="s", num_cores=NC, num_subcores=NS
    )


def _sc_precompute(emb, wfull, bfull):
    N, D = emb.shape
    NP = 2 * D // LANES
    nchunks = N // NCH
    nbat = NCH // 8

    @functools.partial(
        pl.kernel,
        out_type=[
            jax.ShapeDtypeStruct((N, D // 2), jnp.int32),
            jax.ShapeDtypeStruct((N, D // 2), jnp.int32),
        ],
        mesh=_mesh(),
        scratch_types=[
            pltpu.VMEM((NCH, D), jnp.float32),
            pltpu.VMEM((NCH, D // 2), jnp.int32),
            pltpu.VMEM((NCH, D // 2), jnp.int32),
            pltpu.VMEM((D, 2 * D), jnp.float32),
            pltpu.VMEM((2 * D,), jnp.float32),
            pltpu.VMEM((D, D // 2), jnp.int32),
            pltpu.VMEM((D, D // 2), jnp.int32),
        ],
        compiler_params=_SC_PARAMS,
    )
    def k(emb_hbm, w_hbm, b_hbm, a_hbm, bm_hbm, xb, aout, bout, wv, bv,
          wpa, wpb):
        wid = lax.axis_index("s") * NC + lax.axis_index("c")
        pltpu.sync_copy(w_hbm, wv)
        pltpu.sync_copy(b_hbm, bv)
        for kf in range(D):
            pa = plsc.pack(wv[kf, pl.ds(0, LANES)], wv[kf, pl.ds(LANES, LANES)],
                           format=_ILV)
            pb = plsc.pack(wv[kf, pl.ds(2 * LANES, LANES)],
                           wv[kf, pl.ds(3 * LANES, LANES)], format=_ILV)
            wpa[kf, :] = plsc.bitcast(pa, jnp.int32)
            wpb[kf, :] = plsc.bitcast(pb, jnp.int32)
        bpA = plsc.pack(bv[pl.ds(0, LANES)], bv[pl.ds(LANES, LANES)], format=_ILV)
        zpB = jnp.zeros((2 * LANES,), jnp.bfloat16)
        nme = (nchunks - 1 - wid) // NW + 1

        def chunk(i, carry):
            off = (wid + i * NW) * NCH
            pltpu.sync_copy(emb_hbm.at[pl.ds(off, NCH)], xb)

            def bat(bi, c2):
                vbase = bi * 8
                acca = [bpA for _ in range(8)]
                accb = [zpB for _ in range(8)]
                dws = []
                for v in range(8):
                    e0 = xb[vbase + v, pl.ds(0, LANES)]
                    e1 = xb[vbase + v, pl.ds(LANES, LANES)]
                    dws.append((
                        plsc.bitcast(plsc.pack(e0, e0, format=_ILV), jnp.int32),
                        plsc.bitcast(plsc.pack(e1, e1, format=_ILV), jnp.int32),
                    ))
                for kf in range(D):
                    wa = plsc.bitcast(wpa[kf, :], jnp.bfloat16)
                    wb = plsc.bitcast(wpb[kf, :], jnp.bfloat16)
                    for v in range(8):
                        es = plsc.bitcast(
                            _splat(dws[v][kf // LANES], kf % LANES), jnp.bfloat16)
                        acca[v] = acca[v] + es * wa
                        accb[v] = accb[v] + es * wb
                for v in range(8):
                    aout[vbase + v, :] = plsc.bitcast(acca[v], jnp.int32)
                    bout[vbase + v, :] = plsc.bitcast(accb[v], jnp.int32)
                return c2

            lax.fori_loop(0, nbat, bat, 0)
            pltpu.sync_copy(aout, a_hbm.at[pl.ds(off, NCH)])
            pltpu.sync_copy(bout, bm_hbm.at[pl.ds(off, NCH)])
            return carry

        lax.fori_loop(0, nme, chunk, 0)

    return k(emb, wfull, bfull)


def _sc_loss_partials(srcs, dsts, labels, A2, B2, params):
    E = srcs.shape[0]
    NWRD = A2.shape[1]
    per_w = E // NW
    nsub = per_w // SUB
    ngrp = SUB // LANES
    ngath = SUB // GCH
    assert nsub % 2 == 1 and nsub >= 3

    scratch = [
        pltpu.VMEM((SUB,), jnp.int32),
        pltpu.VMEM((SUB,), jnp.int32),
        pltpu.VMEM((SUB,), jnp.int32),
        pltpu.VMEM((SUB, NWRD), jnp.int32),
        pltpu.VMEM((SUB, NWRD), jnp.int32),
        pltpu.VMEM((SUB,), jnp.int32),
        pltpu.VMEM((SUB,), jnp.int32),
        pltpu.VMEM((SUB,), jnp.int32),
        pltpu.VMEM((SUB, NWRD), jnp.int32),
        pltpu.VMEM((SUB, NWRD), jnp.int32),
        pltpu.VMEM((64,), jnp.float32),
        pltpu.VMEM((LANES,), jnp.float32),
        pltpu.SemaphoreType.DMA,
        pltpu.SemaphoreType.DMA,
        pltpu.SemaphoreType.DMA,
        pltpu.SemaphoreType.DMA,
    ]

    @functools.partial(
        pl.kernel,
        out_type=jax.ShapeDtypeStruct((NW, LANES), jnp.float32),
        mesh=_mesh(),
        scratch_types=scratch,
        compiler_params=_SC_PARAMS,
    )
    def k(srcs_hbm, dsts_hbm, labels_hbm, a_hbm, b_hbm, params_hbm, out_hbm,
          sidx0, didx0, lab0, arows0, brows0,
          sidx1, didx1, lab1, arows1, brows1,
          pvv, accv, semi0, semi1, semg0, semg1):
        wid = lax.axis_index("s") * NC + lax.axis_index("c")
        base = wid * per_w
        pltpu.sync_copy(params_hbm, pvv)
        dv0 = pvv[pl.ds(0, LANES)]
        dv1 = pvv[pl.ds(LANES, LANES)]
        t0v = _splat(pvv[pl.ds(2 * LANES, LANES)], 0)
        dpairs = [
            plsc.pack(_splat(dv0, w), _splat(dv1, w), format=_ILV)
            for w in range(NWRD)
        ]
        zero_bf = jnp.zeros((2 * LANES,), jnp.bfloat16)
        iota = lax.iota(jnp.int32, LANES)

        bufs = [
            (sidx0, didx0, lab0, arows0, brows0, semi0, semg0),
            (sidx1, didx1, lab1, arows1, brows1, semi1, semg1),
        ]

        def idx_copies(buf, c):
            sidx, didx, lab, _, _, semi, _ = buf
            off = base + c * SUB
            return [
                pltpu.make_async_copy(srcs_hbm.at[pl.ds(off, SUB)], sidx, semi),
                pltpu.make_async_copy(dsts_hbm.at[pl.ds(off, SUB)], didx, semi),
                pltpu.make_async_copy(labels_hbm.at[pl.ds(off, SUB)], lab, semi),
            ]

        def start_idx(buf, c):
            for cp in idx_copies(buf, c):
                cp.start()

        def wait_idx(buf, c):
            for cp in idx_copies(buf, c):
                cp.wait()

        def g_copies(buf):
            sidx, didx, _, arows, brows, _, semg = buf
            cps = []
            for i in range(ngath):
                sl = pl.ds(i * GCH, GCH)
                cps.append(pltpu.make_async_copy(a_hbm.at[sidx.at[sl]], arows.at[sl], semg))
                cps.append(pltpu.make_async_copy(b_hbm.at[didx.at[sl]], brows.at[sl], semg))
            return cps

        def fire_g(buf):
            for cp in g_copies(buf):
                cp.start()

        def wait_g(buf):
            for cp in g_copies(buf):
                cp.wait()

        def compute(buf, acc):
            _, _, lab, arows, brows, _, _ = buf

            def grp(g, acc2):
                rows = iota + g * LANES
                tl = t0v
                th = jnp.zeros((LANES,), jnp.float32)
                for w in range(NWRD):
                    col = jnp.full((LANES,), w, jnp.int32)
                    wa = plsc.load_gather(arows, [rows, col])
                    wb = plsc.load_gather(brows, [rows, col])
                    u = plsc.bitcast(wa, jnp.bfloat16) + plsc.bitcast(wb, jnp.bfloat16)
                    h = jnp.maximum(u, zero_bf)
                    pr = h * dpairs[w]
                    lo, hi = plsc.unpack(pr, format=_ILV)
                    tl = tl + lo
                    th = th + hi
                t = tl + th
                o = pl.multiple_of(g * LANES, LANES)
                lv = lab[pl.ds(o, LANES)]
                s = (1 - 2 * lv).astype(jnp.float32)
                z = jnp.exp(-jnp.abs(t))
                q = 0.5 * jnp.sign(t) * (1.0 - z) / (1.0 + z)
                q2 = q * q
                gq = LOG2 + q2 * (C1 + q2 * (C2 + q2 * (C3 + q2 * C4)))
                return acc2 + (gq - s * q)

            return lax.fori_loop(0, ngrp, grp, acc)

        start_idx(bufs[0], 0)
        wait_idx(bufs[0], 0)
        fire_g(bufs[0])
        start_idx(bufs[1], 1)

        def body(cc, acc):
            c = 2 * cc
            c2, c3, c4 = c + 1, c + 2, c + 3
            wait_idx(bufs[1], c2)
            fire_g(bufs[1])

            @pl.when(c3 < nsub)
            def _():
                start_idx(bufs[0], c3)

            wait_g(bufs[0])
            acc = compute(bufs[0], acc)

            @pl.when(c3 < nsub)
            def _():
                wait_idx(bufs[0], c3)
                fire_g(bufs[0])

            @pl.when(c4 < nsub)
            def _():
                start_idx(bufs[1], c4)

            wait_g(bufs[1])
            acc = compute(bufs[1], acc)
            return acc

        acc = lax.fori_loop(0, (nsub - 1) // 2, body, jnp.zeros((LANES,), jnp.float32))
        wait_g(bufs[0])
        acc = compute(bufs[0], acc)

        accv[...] = acc
        pltpu.sync_copy(accv, out_hbm.at[wid])

    return k(srcs, dsts, labels, A2, B2, params)


def kernel(edges, labels, word_embeddings, W1, b1, W2, b2):
    D = word_embeddings.shape[1]
    et = edges.T
    srcs = et[0]
    dsts = et[1]
    wfull = jnp.concatenate([W1[:D], W1[D:]], axis=1)
    bfull = jnp.concatenate([b1, jnp.zeros((D,), jnp.float32)])
    A2, B2 = _sc_precompute(word_embeddings, wfull, bfull)
    params = jnp.concatenate([
        W2[:, 0] - W2[:, 1],
        (b2[0] - b2[1])[None],
        jnp.zeros((63 - D,), jnp.float32),
    ])
    partials = _sc_loss_partials(srcs, dsts, labels, A2, B2, params)
    return jnp.sum(partials) / edges.shape[0]


# speedup vs baseline: 5.7511x; 1.0123x over previous
import functools

import jax
import jax.numpy as jnp
from jax import lax
from jax.experimental import pallas as pl
from jax.experimental.pallas import tpu as pltpu
from jax.experimental.pallas import tpu_sc as plsc

NC = 2
NS = 16
NW = NC * NS
LANES = 16
SUB = 400
GCH = 80
NCH = 400

LOG2 = 0.6931471805599453
C1 = 0.5
C2 = -1.0 / 12.0
C3 = 1.0 / 45.0
C4 = -17.0 / 2520.0

_SC_PARAMS = pltpu.CompilerParams(
    needs_layout_passes=False, use_tc_tiling_on_sc=False
)

_ILV = plsc.PackFormat.INTERLEAVED


def _splat(vec, k):
    idx = jnp.full((LANES,), k, jnp.int32)
    return vec.at[idx].get(mode="promise_in_bounds")


def _mesh():
    return plsc.VectorSubcoreMesh(
        core_axis_name="c", subcore_axis_name="s", num_cores=NC, num_subcores=NS
    )


def _sc_precompute(emb, wfull, bfull):
    N, D = emb.shape
    NP = 2 * D // LANES
    nchunks = N // NCH
    nbat = NCH // 8

    @functools.partial(
        pl.kernel,
        out_type=[
            jax.ShapeDtypeStruct((N, D // 2), jnp.int32),
            jax.ShapeDtypeStruct((N, D // 2), jnp.int32),
        ],
        mesh=_mesh(),
        scratch_types=[
            pltpu.VMEM((NCH, D), jnp.float32),
            pltpu.VMEM((NCH, D), jnp.float32),
            pltpu.VMEM((NCH, D // 2), jnp.int32),
            pltpu.VMEM((NCH, D // 2), jnp.int32),
            pltpu.VMEM((D, 2 * D), jnp.float32),
            pltpu.VMEM((2 * D,), jnp.float32),
            pltpu.VMEM((D, D // 2), jnp.int32),
            pltpu.VMEM((D, D // 2), jnp.int32),
            pltpu.SemaphoreType.DMA,
            pltpu.SemaphoreType.DMA,
        ],
        compiler_params=_SC_PARAMS,
    )
    def k(emb_hbm, w_hbm, b_hbm, a_hbm, bm_hbm, xb0, xb1, aout, bout, wv, bv,
          wpa, wpb, semx, semo):
        wid = lax.axis_index("s") * NC + lax.axis_index("c")
        pltpu.sync_copy(w_hbm, wv)
        pltpu.sync_copy(b_hbm, bv)
        for kf in range(D):
            pa = plsc.pack(wv[kf, pl.ds(0, LANES)], wv[kf, pl.ds(LANES, LANES)],
                           format=_ILV)
            pb = plsc.pack(wv[kf, pl.ds(2 * LANES, LANES)],
                           wv[kf, pl.ds(3 * LANES, LANES)], format=_ILV)
            wpa[kf, :] = plsc.bitcast(pa, jnp.int32)
            wpb[kf, :] = plsc.bitcast(pb, jnp.int32)
        bpA = plsc.pack(bv[pl.ds(0, LANES)], bv[pl.ds(LANES, LANES)], format=_ILV)
        zpB = jnp.zeros((2 * LANES,), jnp.bfloat16)
        nme = (nchunks - 1 - wid) // NW + 1

        def in_copy(xb, i):
            off = (wid + i * NW) * NCH
            return pltpu.make_async_copy(emb_hbm.at[pl.ds(off, NCH)], xb, semx)

        def out_copies(i):
            off = (wid + i * NW) * NCH
            return [
                pltpu.make_async_copy(aout, a_hbm.at[pl.ds(off, NCH)], semo),
                pltpu.make_async_copy(bout, bm_hbm.at[pl.ds(off, NCH)], semo),
            ]

        def compute_from(xb):
            def bat(bi, c2):
                vbase = bi * 8
                acca = [[bpA, zpB] for _ in range(8)]
                accb = [[zpB, zpB] for _ in range(8)]
                for h in range(2):
                    dws = []
                    for v in range(8):
                        e = xb[vbase + v, pl.ds(LANES * h, LANES)]
                        dws.append(plsc.bitcast(plsc.pack(e, e, format=_ILV),
                                                jnp.int32))
                    for kk in range(LANES):
                        kf = LANES * h + kk
                        pr = kk % 2
                        wa = plsc.bitcast(wpa[kf, :], jnp.bfloat16)
                        wb = plsc.bitcast(wpb[kf, :], jnp.bfloat16)
                        for v in range(8):
                            es = plsc.bitcast(_splat(dws[v], kk), jnp.bfloat16)
                            acca[v][pr] = acca[v][pr] + es * wa
                            accb[v][pr] = accb[v][pr] + es * wb
                for v in range(8):
                    aout[vbase + v, :] = plsc.bitcast(acca[v][0] + acca[v][1],
                                                     jnp.int32)
                    bout[vbase + v, :] = plsc.bitcast(accb[v][0] + accb[v][1],
                                                     jnp.int32)
                return c2

            lax.fori_loop(0, nbat, bat, 0)

        def chunk(i, carry):
            par = i % 2

            @pl.when(i > 0)
            def _():
                for cp in out_copies(i - 1):
                    cp.wait()

            @pl.when(par == 0)
            def _():
                in_copy(xb0, i).wait()

                @pl.when(i + 1 < nme)
                def _():
                    in_copy(xb1, i + 1).start()

                compute_from(xb0)

            @pl.when(par == 1)
            def _():
                in_copy(xb1, i).wait()

                @pl.when(i + 1 < nme)
                def _():
                    in_copy(xb0, i + 1).start()

                compute_from(xb1)

            for cp in out_copies(i):
                cp.start()
            return carry

        in_copy(xb0, 0).start()
        lax.fori_loop(0, nme, chunk, 0)
        for cp in out_copies(nme - 1):
            cp.wait()

    return k(emb, wfull, bfull)


def _sc_loss_partials(srcs, dsts, labels, A2, B2, params):
    E = srcs.shape[0]
    NWRD = A2.shape[1]
    per_w = E // NW
    nsub = per_w // SUB
    ngrp = SUB // LANES
    ngath = SUB // GCH
    assert nsub % 2 == 1 and nsub >= 3

    scratch = [
        pltpu.VMEM((SUB,), jnp.int32),
        pltpu.VMEM((SUB,), jnp.int32),
        pltpu.VMEM((SUB,), jnp.int32),
        pltpu.VMEM((SUB, NWRD), jnp.int32),
        pltpu.VMEM((SUB, NWRD), jnp.int32),
        pltpu.VMEM((SUB,), jnp.int32),
        pltpu.VMEM((SUB,), jnp.int32),
        pltpu.VMEM((SUB,), jnp.int32),
        pltpu.VMEM((SUB, NWRD), jnp.int32),
        pltpu.VMEM((SUB, NWRD), jnp.int32),
        pltpu.VMEM((64,), jnp.float32),
        pltpu.VMEM((LANES,), jnp.float32),
        pltpu.SemaphoreType.DMA,
        pltpu.SemaphoreType.DMA,
        pltpu.SemaphoreType.DMA,
        pltpu.SemaphoreType.DMA,
    ]

    @functools.partial(
        pl.kernel,
        out_type=jax.ShapeDtypeStruct((NW, LANES), jnp.float32),
        mesh=_mesh(),
        scratch_types=scratch,
        compiler_params=_SC_PARAMS,
    )
    def k(srcs_hbm, dsts_hbm, labels_hbm, a_hbm, b_hbm, params_hbm, out_hbm,
          sidx0, didx0, lab0, arows0, brows0,
          sidx1, didx1, lab1, arows1, brows1,
          pvv, accv, semi0, semi1, semg0, semg1):
        wid = lax.axis_index("s") * NC + lax.axis_index("c")
        base = wid * per_w
        pltpu.sync_copy(params_hbm, pvv)
        dv0 = pvv[pl.ds(0, LANES)]
        dv1 = pvv[pl.ds(LANES, LANES)]
        t0v = _splat(pvv[pl.ds(2 * LANES, LANES)], 0)
        dpairs = [
            plsc.pack(_splat(dv0, w), _splat(dv1, w), format=_ILV)
            for w in range(NWRD)
        ]
        zero_bf = jnp.zeros((2 * LANES,), jnp.bfloat16)
        iota = lax.iota(jnp.int32, LANES)

        bufs = [
            (sidx0, didx0, lab0, arows0, brows0, semi0, semg0),
            (sidx1, didx1, lab1, arows1, brows1, semi1, semg1),
        ]

        def idx_copies(buf, c):
            sidx, didx, lab, _, _, semi, _ = buf
            off = base + c * SUB
            return [
                pltpu.make_async_copy(srcs_hbm.at[pl.ds(off, SUB)], sidx, semi),
                pltpu.make_async_copy(dsts_hbm.at[pl.ds(off, SUB)], didx, semi),
                pltpu.make_async_copy(labels_hbm.at[pl.ds(off, SUB)], lab, semi),
            ]

        def start_idx(buf, c):
            for cp in idx_copies(buf, c):
                cp.start()

        def wait_idx(buf, c):
            for cp in idx_copies(buf, c):
                cp.wait()

        def g_copies(buf):
            sidx, didx, _, arows, brows, _, semg = buf
            cps = []
            for i in range(ngath):
                sl = pl.ds(i * GCH, GCH)
                cps.append(pltpu.make_async_copy(a_hbm.at[sidx.at[sl]], arows.at[sl], semg))
                cps.append(pltpu.make_async_copy(b_hbm.at[didx.at[sl]], brows.at[sl], semg))
            return cps

        def fire_g(buf):
            for cp in g_copies(buf):
                cp.start()

        def wait_g(buf):
            for cp in g_copies(buf):
                cp.wait()

        def compute(buf, acc):
            _, _, lab, arows, brows, _, _ = buf

            def grp(g, acc2):
                rows = iota + g * LANES
                tl = t0v
                th = jnp.zeros((LANES,), jnp.float32)
                for w in range(NWRD):
                    col = jnp.full((LANES,), w, jnp.int32)
                    wa = plsc.load_gather(arows, [rows, col])
                    wb = plsc.load_gather(brows, [rows, col])
                    u = plsc.bitcast(wa, jnp.bfloat16) + plsc.bitcast(wb, jnp.bfloat16)
                    h = jnp.maximum(u, zero_bf)
                    pr = h * dpairs[w]
                    lo, hi = plsc.unpack(pr, format=_ILV)
                    tl = tl + lo
                    th = th + hi
                t = tl + th
                o = pl.multiple_of(g * LANES, LANES)
                lv = lab[pl.ds(o, LANES)]
                s = (1 - 2 * lv).astype(jnp.float32)
                z = jnp.exp(-jnp.abs(t))
                q = 0.5 * jnp.sign(t) * (1.0 - z) / (1.0 + z)
                q2 = q * q
                gq = LOG2 + q2 * (C1 + q2 * (C2 + q2 * (C3 + q2 * C4)))
                return acc2 + (gq - s * q)

            return lax.fori_loop(0, ngrp, grp, acc)

        start_idx(bufs[0], 0)
        wait_idx(bufs[0], 0)
        fire_g(bufs[0])
        start_idx(bufs[1], 1)

        def body(cc, acc):
            c = 2 * cc
            c2, c3, c4 = c + 1, c + 2, c + 3
            wait_idx(bufs[1], c2)
            fire_g(bufs[1])

            @pl.when(c3 < nsub)
            def _():
                start_idx(bufs[0], c3)

            wait_g(bufs[0])
            acc = compute(bufs[0], acc)

            @pl.when(c3 < nsub)
            def _():
                wait_idx(bufs[0], c3)
                fire_g(bufs[0])

            @pl.when(c4 < nsub)
            def _():
                start_idx(bufs[1], c4)

            wait_g(bufs[1])
            acc = compute(bufs[1], acc)
            return acc

        acc = lax.fori_loop(0, (nsub - 1) // 2, body, jnp.zeros((LANES,), jnp.float32))
        wait_g(bufs[0])
        acc = compute(bufs[0], acc)

        accv[...] = acc
        pltpu.sync_copy(accv, out_hbm.at[wid])

    return k(srcs, dsts, labels, A2, B2, params)


def kernel(edges, labels, word_embeddings, W1, b1, W2, b2):
    D = word_embeddings.shape[1]
    et = edges.T
    srcs = et[0]
    dsts = et[1]
    wfull = jnp.concatenate([W1[:D], W1[D:]], axis=1)
    bfull = jnp.concatenate([b1, jnp.zeros((D,), jnp.float32)])
    A2, B2 = _sc_precompute(word_embeddings, wfull, bfull)
    params = jnp.concatenate([
        W2[:, 0] - W2[:, 1],
        (b2[0] - b2[1])[None],
        jnp.zeros((63 - D,), jnp.float32),
    ])
    partials = _sc_loss_partials(srcs, dsts, labels, A2, B2, params)
    return jnp.sum(partials) / edges.shape[0]
